# baseline (speedup 1.0000x reference)
import functools

import jax
import jax.numpy as jnp
from jax import lax
from jax.experimental import pallas as pl
from jax.experimental.pallas import tpu as pltpu
from jax.experimental.pallas import tpu_sc as plsc

_VMEM_LIMIT = pltpu.CompilerParams(vmem_limit_bytes=63 * 1024 * 1024)

_NW = 32
_K = 8
_CSPLIT = 4992
_NPAD = 10240


def _sc_deg(adj):
    n = adj.shape[0]
    base_rows = 312
    nchunk = base_rows // _K
    wa = _CSPLIT
    wb = n - _CSPLIT

    mesh = plsc.VectorSubcoreMesh(core_axis_name="c", subcore_axis_name="s")

    @functools.partial(
        pl.kernel,
        out_type=jax.ShapeDtypeStruct((_NW * _NPAD,), jnp.float32),
        mesh=mesh,
        scratch_types=[
            pltpu.VMEM((_K, wa), jnp.float32),
            pltpu.VMEM((_K, wb), jnp.float32),
            pltpu.VMEM((_NPAD,), jnp.float32),
            pltpu.SemaphoreType.DMA,
            pltpu.SemaphoreType.DMA,
        ],
    )
    def k(adj_hbm, out_hbm, buf_a, buf_b, acc, sem_a, sem_b):
        c = lax.axis_index("c")
        s = lax.axis_index("s")
        wid = c * 16 + s
        row0 = wid * base_rows + 8 * jnp.minimum(wid, 2)

        def zero_body(i, _):
            acc[pl.ds(i * 16, 16)] = jnp.zeros((16,), jnp.float32)
            return 0

        lax.fori_loop(0, _NPAD // 16, zero_body, 0)

        def accum_a(i, _):
            v = acc[pl.ds(i * 16, 16)]
            for r in range(_K):
                v = v + buf_a[r, pl.ds(i * 16, 16)]
            acc[pl.ds(i * 16, 16)] = v
            return 0

        def accum_b(i, _):
            v = acc[pl.ds(_CSPLIT + i * 16, 16)]
            for r in range(_K):
                v = v + buf_b[r, pl.ds(i * 16, 16)]
            acc[pl.ds(_CSPLIT + i * 16, 16)] = v
            return 0

        def chunk(start):
            cp_a = pltpu.make_async_copy(
                adj_hbm.at[pl.ds(start, _K), pl.ds(0, wa)], buf_a, sem_a)
            cp_b = pltpu.make_async_copy(
                adj_hbm.at[pl.ds(start, _K), pl.ds(_CSPLIT, wb)], buf_b,
                sem_b)
            cp_a.start()
            cp_b.start()
            cp_a.wait()
            lax.fori_loop(0, wa // 16, accum_a, 0)
            cp_b.wait()
            lax.fori_loop(0, wb // 16, accum_b, 0)

        def chunk_body(g, _):
            chunk(row0 + g * _K)
            return 0

        lax.fori_loop(0, nchunk, chunk_body, 0)

        @pl.when(wid < 2)
        def _():
            chunk(row0 + base_rows)

        pltpu.sync_copy(acc, out_hbm.at[pl.ds(wid * _NPAD, _NPAD)])

    return k(adj)


def _prep_kernel(adj_ref, part_ref, mask_ref, dinv_ref, *, nblk, blk, n):
    j = pl.program_id(0)
    a = adj_ref[...]
    mask_ref[...] = a.astype(jnp.int8)

    @pl.when(j == 0)
    def _():
        deg = 1.0 + jnp.sum(part_ref[:, :n], axis=0, keepdims=True)
        dinv_ref[...] = jnp.where(deg > 0, jax.lax.rsqrt(deg), 0.0)


def _gcn_kernel(mask_ref, x_ref, w_ref, b_ref, dinv_ref, out_ref,
                u_ref, dcol_ref, *, nblk, blk, relu, logsm, temp):
    j = pl.program_id(0)

    @pl.when(j == 0)
    def _():
        one = jnp.ones((1, 1), dtype=jnp.float32)
        dcol_ref[...] = jax.lax.dot_general(
            dinv_ref[...], one, (((0,), (0,)), ((), ())),
            preferred_element_type=jnp.float32)
        u = dcol_ref[...] * jnp.dot(
            x_ref[...], w_ref[...], preferred_element_type=jnp.float32)
        u_ref[...] = u.astype(jnp.bfloat16)
        out_ref[...] = jnp.zeros_like(out_ref)

    u_blk = u_ref[pl.ds(j * blk, blk), :]
    out_ref[...] += jax.lax.dot_general(
        mask_ref[...].astype(jnp.bfloat16), u_blk, (((0,), (0,)), ((), ())),
        preferred_element_type=jnp.float32)

    @pl.when(j == nblk - 1)
    def _():
        v = dcol_ref[...] * (out_ref[...] + u_ref[...].astype(jnp.float32))
        v = v + b_ref[...]
        if relu:
            v = jnp.maximum(v, 0.0)
        if logsm:
            t = v * (1.0 / temp)
            m = jnp.max(t, axis=1, keepdims=True)
            sh = t - m
            v = sh - jnp.log(jnp.sum(jnp.exp(sh), axis=1, keepdims=True))
        out_ref[...] = v


def kernel(x, adj, W1, b1, W2, b2):
    n = adj.shape[0]
    blk = 400
    nblk = n // blk

    partials = _sc_deg(adj).reshape(_NW, _NPAD)

    mask, dinv = pl.pallas_call(
        functools.partial(_prep_kernel, nblk=nblk, blk=blk, n=n),
        grid=(nblk,),
        in_specs=[
            pl.BlockSpec((blk, n), lambda j: (j, 0)),
            pl.BlockSpec((_NW, _NPAD), lambda j: (0, 0)),
        ],
        out_specs=[
            pl.BlockSpec((blk, n), lambda j: (j, 0)),
            pl.BlockSpec((1, n), lambda j: (0, 0)),
        ],
        out_shape=[
            jax.ShapeDtypeStruct((n, n), jnp.int8),
            jax.ShapeDtypeStruct((1, n), jnp.float32),
        ],
        compiler_params=_VMEM_LIMIT,
    )(adj, partials)

    lblk = 400
    lnblk = n // lblk

    def layer(h, w, b, relu, logsm, temp):
        f = w.shape[1]
        return pl.pallas_call(
            functools.partial(_gcn_kernel, nblk=lnblk, blk=lblk, relu=relu,
                              logsm=logsm, temp=temp),
            grid=(lnblk,),
            in_specs=[
                pl.BlockSpec((lblk, n), lambda j: (j, 0)),
                pl.BlockSpec((n, h.shape[1]), lambda j: (0, 0)),
                pl.BlockSpec(w.shape, lambda j: (0, 0)),
                pl.BlockSpec((1, f), lambda j: (0, 0)),
                pl.BlockSpec((1, n), lambda j: (0, 0)),
            ],
            out_specs=pl.BlockSpec((n, f), lambda j: (0, 0)),
            out_shape=jax.ShapeDtypeStruct((n, f), jnp.float32),
            scratch_shapes=[
                pltpu.VMEM((n, f), jnp.bfloat16),
                pltpu.VMEM((n, 1), jnp.float32),
            ],
            compiler_params=_VMEM_LIMIT,
        )(mask, h, w, b.reshape(1, f), dinv)

    h1 = layer(x, W1, b1, relu=True, logsm=False, temp=1.0)
    out = layer(h1, W2, b2, relu=False, logsm=True, temp=0.2)
    return out


# speedup vs baseline: 2.7735x; 2.7735x over previous
import functools

import jax
import jax.numpy as jnp
from jax.experimental import pallas as pl
from jax.experimental.pallas import tpu as pltpu

_VMEM_LIMIT = pltpu.CompilerParams(vmem_limit_bytes=63 * 1024 * 1024)


def _pick_blk(n):
    for blk in (400, 80, 16):
        if n % blk == 0:
            return blk
    return n


def _prep_kernel(adj_ref, mask_ref, dinv_ref, *, nblk, blk):
    j = pl.program_id(0)
    a = adj_ref[...]
    mask_ref[...] = a.astype(jnp.int8)

    @pl.when(j == 0)
    def _():
        dinv_ref[...] = jnp.ones_like(dinv_ref)

    ones = jnp.ones((1, blk), dtype=jnp.float32)
    dinv_ref[...] += jax.lax.dot_general(
        ones, a, (((1,), (0,)), ((), ())),
        preferred_element_type=jnp.float32)

    @pl.when(j == nblk - 1)
    def _():
        d = dinv_ref[...]
        dinv_ref[...] = jnp.where(d > 0, jax.lax.rsqrt(d), 0.0)


def _gcn_kernel(mask_ref, x_ref, w_ref, b_ref, dinv_ref, out_ref,
                u_ref, dcol_ref, *, nblk, blk, relu, logsm, temp):
    j = pl.program_id(0)

    @pl.when(j == 0)
    def _():
        one = jnp.ones((1, 1), dtype=jnp.float32)
        dcol_ref[...] = jax.lax.dot_general(
            dinv_ref[...], one, (((0,), (0,)), ((), ())),
            preferred_element_type=jnp.float32)
        u = dcol_ref[...] * jnp.dot(
            x_ref[...], w_ref[...], preferred_element_type=jnp.float32)
        u_ref[...] = u.astype(jnp.bfloat16)
        out_ref[...] = jnp.zeros_like(out_ref)

    u_blk = u_ref[pl.ds(j * blk, blk), :]
    out_ref[...] += jax.lax.dot_general(
        mask_ref[...].astype(jnp.bfloat16), u_blk, (((0,), (0,)), ((), ())),
        preferred_element_type=jnp.float32)

    @pl.when(j == nblk - 1)
    def _():
        v = dcol_ref[...] * (out_ref[...] + u_ref[...].astype(jnp.float32))
        v = v + b_ref[...]
        if relu:
            v = jnp.maximum(v, 0.0)
        if logsm:
            t = v * (1.0 / temp)
            m = jnp.max(t, axis=1, keepdims=True)
            s = t - m
            v = s - jnp.log(jnp.sum(jnp.exp(s), axis=1, keepdims=True))
        out_ref[...] = v


def kernel(x, adj, W1, b1, W2, b2):
    n = adj.shape[0]
    blk = _pick_blk(n)
    nblk = n // blk

    mask, dinv = pl.pallas_call(
        functools.partial(_prep_kernel, nblk=nblk, blk=blk),
        grid=(nblk,),
        in_specs=[pl.BlockSpec((blk, n), lambda j: (j, 0))],
        out_specs=[
            pl.BlockSpec((blk, n), lambda j: (j, 0)),
            pl.BlockSpec((1, n), lambda j: (0, 0)),
        ],
        out_shape=[
            jax.ShapeDtypeStruct((n, n), jnp.int8),
            jax.ShapeDtypeStruct((1, n), jnp.float32),
        ],
        compiler_params=_VMEM_LIMIT,
    )(adj)

    lblk = 1000 if n % 1000 == 0 else blk
    lnblk = n // lblk

    def layer(h, w, b, relu, logsm, temp):
        f = w.shape[1]
        return pl.pallas_call(
            functools.partial(_gcn_kernel, nblk=lnblk, blk=lblk, relu=relu,
                              logsm=logsm, temp=temp),
            grid=(lnblk,),
            in_specs=[
                pl.BlockSpec((lblk, n), lambda j: (j, 0)),
                pl.BlockSpec((n, h.shape[1]), lambda j: (0, 0)),
                pl.BlockSpec(w.shape, lambda j: (0, 0)),
                pl.BlockSpec((1, f), lambda j: (0, 0)),
                pl.BlockSpec((1, n), lambda j: (0, 0)),
            ],
            out_specs=pl.BlockSpec((n, f), lambda j: (0, 0)),
            out_shape=jax.ShapeDtypeStruct((n, f), jnp.float32),
            scratch_shapes=[
                pltpu.VMEM((n, f), jnp.bfloat16),
                pltpu.VMEM((n, 1), jnp.float32),
            ],
            compiler_params=_VMEM_LIMIT,
        )(mask, h, w, b.reshape(1, f), dinv)

    h1 = layer(x, W1, b1, relu=True, logsm=False, temp=1.0)
    out = layer(h1, W2, b2, relu=False, logsm=True, temp=0.2)
    return out
